# baseline (speedup 1.0000x reference)
import functools

import jax
import jax.numpy as jnp
from jax import lax
from jax.experimental import pallas as pl
from jax.experimental.pallas import tpu as pltpu
from jax.experimental.pallas import tpu_sc as plsc

_LANE = 16
_NC, _NS = 2, 16
_NW = _NC * _NS
_PADW = 128


@functools.lru_cache(maxsize=None)
def _build(batch, seq_len, emb_dim):
    n_tok = batch * seq_len
    tok_per_w = n_tok // _NW
    tok_chunk = seq_len
    rows_per_w = batch // _NW
    n_chunks = tok_per_w // tok_chunk
    n_groups = emb_dim // _LANE
    subs = []
    off = 0
    while off < tok_chunk:
        sz = min(128, tok_chunk - off)
        subs.append((off, sz))
        off += sz

    mesh = plsc.VectorSubcoreMesh(core_axis_name="c", subcore_axis_name="s")

    @functools.partial(
        pl.kernel,
        out_type=jax.ShapeDtypeStruct((batch, seq_len, _PADW), jnp.float32),
        mesh=mesh,
        scratch_types=[
            pltpu.VMEM((tok_per_w,), jnp.int32),
            pltpu.VMEM((2, tok_chunk, emb_dim), jnp.float32),
            pltpu.VMEM((2, tok_chunk, _PADW), jnp.float32),
            pltpu.VMEM((seq_len, emb_dim), jnp.float32),
        ]
        + [pltpu.SemaphoreType.DMA] * 4,
        compiler_params=pltpu.CompilerParams(use_tc_tiling_on_sc=False),
    )
    def emb_kernel(x_hbm, tok_hbm, pos_hbm, out_hbm, idx_v, gath_v, pad_v,
                   pos_v, *sems):
        sem_g = sems[:2]
        sem_o = sems[2:]
        wid = lax.axis_index("s") * _NC + lax.axis_index("c")
        base = wid * tok_per_w
        brow0 = wid * rows_per_w
        pltpu.sync_copy(pos_hbm.at[pl.ds(0, seq_len)], pos_v)
        pltpu.sync_copy(x_hbm.at[pl.ds(base, tok_per_w)], idx_v)

        def fire_gather(c, slot):
            for so, sz in subs:
                pltpu.async_copy(
                    tok_hbm.at[idx_v.at[pl.ds(c * tok_chunk + so, sz)]],
                    gath_v.at[slot].at[pl.ds(so, sz)],
                    sem_g[slot],
                )

        def wait_gather(c, slot):
            for so, sz in subs:
                pltpu.make_async_copy(
                    tok_hbm.at[idx_v.at[pl.ds(c * tok_chunk + so, sz)]],
                    gath_v.at[slot].at[pl.ds(so, sz)],
                    sem_g[slot],
                ).wait()

        def fire_scatter(c, slot):
            pltpu.async_copy(
                pad_v.at[slot],
                out_hbm.at[brow0 + c],
                sem_o[slot],
            )

        def wait_scatter(c, slot):
            pltpu.make_async_copy(
                pad_v.at[slot],
                out_hbm.at[brow0 + c],
                sem_o[slot],
            ).wait()

        fire_gather(0, 0)
        fire_gather(1, 1)

        def body(q, carry):
            for j in range(2):
                c = q * 2 + j
                wait_gather(c, j)

                @pl.when(c >= 2)
                def _():
                    wait_scatter(c - 2, j)

                def add_body(t, _):
                    for d in range(n_groups):
                        sl = pl.ds(d * _LANE, _LANE)
                        pad_v[j, t, sl] = gath_v[j, t, sl] + pos_v[t, sl]
                    return 0

                lax.fori_loop(0, tok_chunk, add_body, 0)
                fire_scatter(c, j)

                @pl.when(c + 2 < n_chunks)
                def _():
                    fire_gather(c + 2, j)
            return carry

        lax.fori_loop(0, n_chunks // 2, body, 0)
        wait_scatter(n_chunks - 2, 0)
        wait_scatter(n_chunks - 1, 1)

    return emb_kernel


@jax.jit
def kernel(x, token_table, pos_table):
    batch, seq_len = x.shape
    emb_dim = token_table.shape[1]
    xf = x.reshape(-1).astype(jnp.int32)
    out = _build(batch, seq_len, emb_dim)(xf, token_table, pos_table)
    return out[:, :, :emb_dim]
